# baseline (speedup 1.0000x reference)
import functools

import jax
import jax.numpy as jnp
from jax import lax
from jax.experimental import pallas as pl
from jax.experimental.pallas import tpu as pltpu
from jax.experimental.pallas import tpu_sc as plsc

f32 = jnp.float32
i32 = jnp.int32

N = 10000
NTILES = 32
NT = 320
NPAD = NTILES * NT
HID = 64
EB = 128
NG = 64

_HIGH = jax.lax.Precision.HIGHEST


def _proj0_body(x_ref, w_ref, b_ref, out_ref):
    out_ref[...] = jnp.dot(x_ref[...], w_ref[...], precision=_HIGH,
                           preferred_element_type=f32) + b_ref[...]


def _proj_relu_body(prev_ref, bias_ref, w_ref, b_ref, out_ref):
    h = jnp.maximum(prev_ref[...] + bias_ref[...], 0.0)
    out_ref[...] = jnp.dot(h, w_ref[...], precision=_HIGH,
                           preferred_element_type=f32) + b_ref[...]


def _final_body(prev_ref, bias_ref, batch_ref, fcw_ref, fcb_ref, out_ref):
    h = prev_ref[...] + bias_ref[...]
    g = lax.broadcasted_iota(i32, (NG, NPAD), 0)
    oht = (batch_ref[...] == g).astype(f32)
    pooled = jnp.dot(oht, h, precision=_HIGH, preferred_element_type=f32)
    out_ref[...] = jnp.dot(pooled, fcw_ref[...], precision=_HIGH,
                           preferred_element_type=f32) + fcb_ref[...]


def _sc_layer(tab, src_s, dst_s, ea4, tp, par):
    mesh = plsc.VectorSubcoreMesh(core_axis_name="c", subcore_axis_name="s")

    @functools.partial(
        pl.kernel,
        mesh=mesh,
        out_type=jax.ShapeDtypeStruct((NPAD * HID,), f32),
        scratch_types=[
            pltpu.VMEM((NT, 128), f32),
            pltpu.VMEM((EB, 128), f32),
            pltpu.VMEM((EB, 128), f32),
            pltpu.VMEM((128,), i32),
            pltpu.VMEM((128,), i32),
            pltpu.VMEM((EB + 16,), i32),
            pltpu.VMEM((EB + 16,), i32),
            pltpu.VMEM((4 * EB + 16,), f32),
            pltpu.VMEM((4 * EB + 16,), f32),
            pltpu.VMEM((4, 64), f32),
            pltpu.VMEM((48,), i32),
            pltpu.VMEM((NT * HID,), f32),
            pltpu.SemaphoreType.DMA,
            pltpu.SemaphoreType.DMA,
        ],
    )
    def k(tab_hbm, src_hbm, dst_hbm, ea_hbm, tp_hbm, par_hbm, out_hbm,
          xlxr, rows0, rows1, srcv0, srcv1, dstv0, dstv1, eav0, eav1,
          parv, tpv, outb, semA, semB):
        cid = lax.axis_index("c")
        sid = lax.axis_index("s")
        wid = sid * 2 + cid
        d0 = pl.multiple_of(wid * NT, 8)

        pltpu.sync_copy(tp_hbm, tpv)
        pltpu.sync_copy(par_hbm, parv)
        pltpu.sync_copy(tab_hbm.at[pl.ds(d0, NT), :], xlxr)

        tpw = tpv[pl.ds(wid, 16)]
        e0 = tpw[0]
        e1 = tpw[1]
        e0al = pl.multiple_of((e0 // 8) * 8, 8)
        nblk = (e1 - e0al + EB - 1) // EB

        W0 = [parv[0, pl.ds(kk * 16, 16)] for kk in range(4)]
        W1 = [parv[1, pl.ds(kk * 16, 16)] for kk in range(4)]
        W2 = [parv[2, pl.ds(kk * 16, 16)] for kk in range(4)]
        AT = [parv[3, pl.ds(kk * 16, 16)] for kk in range(4)]

        def init_body(d, c):
            for kk in range(4):
                outb[pl.ds(d * HID + kk * 16, 16)] = xlxr[d, pl.ds(kk * 16, 16)]
            return c
        lax.fori_loop(0, NT, init_body, i32(0))

        rows = (rows0, rows1)
        srcv = (srcv0, srcv1)
        dstv = (dstv0, dstv1)
        eav = (eav0, eav1)
        sems = (semA, semB)

        def stage_and_fire(blk, slot):
            base = pl.multiple_of(e0al + blk * EB, 8)
            pltpu.sync_copy(src_hbm.at[pl.ds(base, EB)], srcv[slot])
            pltpu.sync_copy(dst_hbm.at[pl.ds(base, EB)],
                            dstv[slot].at[pl.ds(0, EB)])
            pltpu.sync_copy(ea_hbm.at[pl.ds(base * 4, 4 * EB)],
                            eav[slot].at[pl.ds(0, 4 * EB)])
            pltpu.async_copy(tab_hbm.at[srcv[slot]], rows[slot], sems[slot])

        @pl.when(nblk > 0)
        def _():
            stage_and_fire(0, 0)

        neg = jnp.full((16,), -1e30, f32)
        zero = jnp.zeros((16,), f32)
        one = jnp.ones((16,), f32)
        io16 = lax.iota(i32, 16)

        def allsum(v):
            for sh in (8, 4, 2, 1):
                v = v + v.at[io16 ^ sh].get(mode="promise_in_bounds",
                                            unique_indices=True)
            return v

        def finalize(cur_d, cnt, a0, a1, a2, mx_v, den_v, accs):
            dloc = cur_d - d0
            inv_v = 1.0 / jnp.full((16,), jnp.maximum(cnt, 1).astype(f32), f32)
            b0 = a0 * inv_v
            b1 = a1 * inv_v
            b2 = a2 * inv_v
            part = zero
            xld = [xlxr[dloc, pl.ds(kk * 16, 16)] for kk in range(4)]
            for kk in range(4):
                m = (xld[kk] + xlxr[dloc, pl.ds(64 + kk * 16, 16)]
                     + b0 * W0[kk] + b1 * W1[kk] + b2 * W2[kk])
                m = jnp.where(m > 0, m, 0.2 * m)
                part = part + m * AT[kk]
            lg_v = allsum(part)
            nmx = jnp.maximum(mx_v, lg_v)
            s = jnp.exp(mx_v - nmx)
            w = jnp.exp(lg_v - nmx)
            den = den_v * s + w
            for kk in range(4):
                out_k = (accs[kk] * s + w * xld[kk]) / den
                outb[pl.ds(dloc * HID + kk * 16, 16)] = out_k

        def edge_range(lo, hi, base, slot, carry):
            def edge_body(e, c):
                (cur_d, cnt, a0, a1, a2, mx_v, den_v,
                 ac0, ac1, ac2, ac3) = c
                accs = (ac0, ac1, ac2, ac3)
                j = e - base
                dnew = dstv[slot][pl.ds(j, 16)][0]
                bnd = dnew != cur_d

                @pl.when(bnd & (cur_d >= 0))
                def _():
                    finalize(cur_d, cnt, a0, a1, a2, mx_v, den_v, accs)

                ev = eav[slot][pl.ds(4 * j, 16)]
                c0 = ev[0]
                c1 = ev[1]
                c2 = ev[2]
                cnt = jnp.where(bnd, 1, cnt + 1)
                a0 = jnp.where(bnd, c0, a0 + c0)
                a1 = jnp.where(bnd, c1, a1 + c1)
                a2 = jnp.where(bnd, c2, a2 + c2)
                mx_v = jnp.where(bnd, neg, mx_v)
                den_v = jnp.where(bnd, zero, den_v)
                accs = tuple(jnp.where(bnd, zero, a) for a in accs)
                dloc = dnew - d0

                part = zero
                rws = []
                for kk in range(4):
                    row = rows[slot][j, pl.ds(kk * 16, 16)]
                    rws.append(row)
                    m = (row + xlxr[dloc, pl.ds(64 + kk * 16, 16)]
                         + c0 * W0[kk] + c1 * W1[kk] + c2 * W2[kk])
                    m = jnp.where(m > 0, m, 0.2 * m)
                    part = part + m * AT[kk]
                lg_v = allsum(part)
                nmx = jnp.maximum(mx_v, lg_v)
                t = jnp.exp(jnp.minimum(mx_v, lg_v) - nmx)
                gt = lg_v > mx_v
                s = jnp.where(gt, t, one)
                w = jnp.where(gt, one, t)
                den_v = den_v * s + w
                accs = tuple(accs[kk] * s + w * rws[kk] for kk in range(4))
                return (dnew, cnt, a0, a1, a2, nmx, den_v) + accs

            return plsc.parallel_loop(lo, hi, unroll=2, carry=carry)(edge_body)

        carry0 = (i32(-1), i32(0), f32(0), f32(0), f32(0),
                  neg, zero, zero, zero, zero, zero)

        def pair_body(bp, carry):
            for phase in range(2):
                blk = 2 * bp + phase
                slot = phase
                base = e0al + blk * EB

                @pl.when(blk + 1 < nblk)
                def _():
                    stage_and_fire(blk + 1, 1 - slot)

                @pl.when(blk < nblk)
                def _():
                    pltpu.make_async_copy(tab_hbm.at[srcv[slot]],
                                          rows[slot], sems[slot]).wait()

                lo = jnp.maximum(e0, base)
                hi = jnp.minimum(e1, base + EB)
                hi = jnp.maximum(hi, lo)
                carry = edge_range(lo, hi, base, slot, carry)
            return carry

        npairs = (nblk + 1) // 2
        carry = lax.fori_loop(0, npairs, pair_body, carry0)

        (cur_d, cnt, a0, a1, a2, mx_v, den_v, ac0, ac1, ac2, ac3) = carry

        @pl.when(cur_d >= 0)
        def _():
            finalize(cur_d, cnt, a0, a1, a2, mx_v, den_v,
                     (ac0, ac1, ac2, ac3))

        pltpu.sync_copy(outb, out_hbm.at[pl.ds(pl.multiple_of(d0 * HID, 8), NT * HID)])

    return k(tab, src_s, dst_s, ea4, tp, par)


def kernel(x, edge_index, edge_attr, batch, params, fc_w, fc_b):
    E = edge_index.shape[1]
    src = edge_index[0].astype(i32)
    dst = edge_index[1].astype(i32)

    perm = jnp.argsort(dst)
    src_s = src[perm]
    dst_s = dst[perm]
    ea_s = edge_attr[perm]

    EPAD = E + 2 * EB
    src_p = jnp.zeros((EPAD,), i32).at[:E].set(src_s)
    dst_p = jnp.full((EPAD,), -7, i32).at[:E].set(dst_s)
    ea_p = jnp.zeros((EPAD, 4), f32).at[:E, :3].set(ea_s).reshape(-1)

    bounds = jnp.minimum(jnp.arange(33, dtype=i32) * NT, N)
    tp = jnp.searchsorted(dst_s, bounds).astype(i32)
    tp = jnp.concatenate([tp, jnp.full((15,), E, i32)])

    x_pad = jnp.zeros((NPAD, HID), f32).at[:N, :x.shape[1]].set(x)
    batch_pad = jnp.full((1, NPAD), NG, i32).at[0, :N].set(batch.astype(i32))

    proj0 = pl.pallas_call(
        _proj0_body,
        out_shape=jax.ShapeDtypeStruct((NPAD, 2 * HID), f32),
    )
    projr = pl.pallas_call(
        _proj_relu_body,
        out_shape=jax.ShapeDtypeStruct((NPAD, 2 * HID), f32),
    )
    final = pl.pallas_call(
        _final_body,
        out_shape=jax.ShapeDtypeStruct((NG, fc_w.shape[1]), f32),
    )

    prev = None
    for li, p in enumerate(params):
        din = p['Wl'].shape[0]
        wl = jnp.zeros((HID, HID), f32).at[:din].set(p['Wl'])
        wr = jnp.zeros((HID, HID), f32).at[:din].set(p['Wr'])
        w2 = jnp.concatenate([wl, wr], axis=1)
        b2 = jnp.concatenate([p['bl'], p['br']])
        if li == 0:
            tab = proj0(x_pad, w2, b2)
        else:
            tab = projr(prev, params[li - 1]['bias'], w2, b2)
        par = jnp.concatenate([p['We'], p['att'][None, :]], axis=0)
        out_flat = _sc_layer(tab, src_p, dst_p, ea_p, tp, par)
        prev = out_flat.reshape(NPAD, HID)

    return final(prev, params[-1]['bias'], batch_pad, fc_w, fc_b)


# speedup vs baseline: 1.1206x; 1.1206x over previous
import functools

import jax
import jax.numpy as jnp
from jax import lax
from jax.experimental import pallas as pl
from jax.experimental.pallas import tpu as pltpu
from jax.experimental.pallas import tpu_sc as plsc

f32 = jnp.float32
i32 = jnp.int32

N = 10000
NTILES = 32
NT = 320
NPAD = NTILES * NT
HID = 64
EB = 128
NG = 64

_HIGH = jax.lax.Precision.HIGHEST


def _proj0_body(x_ref, w_ref, b_ref, out_ref):
    out_ref[...] = jnp.dot(x_ref[...], w_ref[...], precision=_HIGH,
                           preferred_element_type=f32) + b_ref[...]


def _proj_relu_body(prev_ref, bias_ref, w_ref, b_ref, out_ref):
    h = jnp.maximum(prev_ref[...] + bias_ref[...], 0.0)
    out_ref[...] = jnp.dot(h, w_ref[...], precision=_HIGH,
                           preferred_element_type=f32) + b_ref[...]


def _final_body(prev_ref, bias_ref, batch_ref, fcw_ref, fcb_ref, out_ref):
    h = prev_ref[...] + bias_ref[...]
    g = lax.broadcasted_iota(i32, (NG, NPAD), 0)
    oht = (batch_ref[...] == g).astype(f32)
    pooled = jnp.dot(oht, h, precision=_HIGH, preferred_element_type=f32)
    out_ref[...] = jnp.dot(pooled, fcw_ref[...], precision=_HIGH,
                           preferred_element_type=f32) + fcb_ref[...]


def _sc_layer(tab, src_s, dst_s, ea4, tp, par):
    mesh = plsc.VectorSubcoreMesh(core_axis_name="c", subcore_axis_name="s")

    @functools.partial(
        pl.kernel,
        mesh=mesh,
        out_type=jax.ShapeDtypeStruct((NPAD * HID,), f32),
        scratch_types=[
            pltpu.VMEM((NT, 128), f32),
            pltpu.VMEM((EB, 128), f32),
            pltpu.VMEM((EB, 128), f32),
            pltpu.VMEM((128,), i32),
            pltpu.VMEM((128,), i32),
            pltpu.VMEM((EB + 16,), i32),
            pltpu.VMEM((EB + 16,), i32),
            pltpu.VMEM((4 * EB + 16,), f32),
            pltpu.VMEM((4 * EB + 16,), f32),
            pltpu.VMEM((4, 64), f32),
            pltpu.VMEM((48,), i32),
            pltpu.VMEM((NT * HID,), f32),
            pltpu.SemaphoreType.DMA,
            pltpu.SemaphoreType.DMA,
        ],
    )
    def k(tab_hbm, src_hbm, dst_hbm, ea_hbm, tp_hbm, par_hbm, out_hbm,
          xlxr, rows0, rows1, srcv0, srcv1, dstv0, dstv1, eav0, eav1,
          parv, tpv, outb, semA, semB):
        cid = lax.axis_index("c")
        sid = lax.axis_index("s")
        wid = sid * 2 + cid
        d0 = pl.multiple_of(wid * NT, 8)

        pltpu.sync_copy(tp_hbm, tpv)
        pltpu.sync_copy(par_hbm, parv)
        pltpu.sync_copy(tab_hbm.at[pl.ds(d0, NT), :], xlxr)

        tpw = tpv[pl.ds(wid, 16)]
        e0 = tpw[0]
        e1 = tpw[1]
        e0al = pl.multiple_of((e0 // 8) * 8, 8)
        nblk = (e1 - e0al + EB - 1) // EB

        W0 = [parv[0, pl.ds(kk * 16, 16)] for kk in range(4)]
        W1 = [parv[1, pl.ds(kk * 16, 16)] for kk in range(4)]
        W2 = [parv[2, pl.ds(kk * 16, 16)] for kk in range(4)]
        AT = [parv[3, pl.ds(kk * 16, 16)] for kk in range(4)]

        def init_body(d, c):
            for kk in range(4):
                outb[pl.ds(d * HID + kk * 16, 16)] = xlxr[d, pl.ds(kk * 16, 16)]
            return c
        lax.fori_loop(0, NT, init_body, i32(0))

        rows = (rows0, rows1)
        srcv = (srcv0, srcv1)
        dstv = (dstv0, dstv1)
        eav = (eav0, eav1)
        sems = (semA, semB)

        def stage_and_fire(blk, slot):
            base = pl.multiple_of(e0al + blk * EB, 8)
            pltpu.sync_copy(src_hbm.at[pl.ds(base, EB)], srcv[slot])
            pltpu.sync_copy(dst_hbm.at[pl.ds(base, EB)],
                            dstv[slot].at[pl.ds(0, EB)])
            pltpu.sync_copy(ea_hbm.at[pl.ds(base * 4, 4 * EB)],
                            eav[slot].at[pl.ds(0, 4 * EB)])
            pltpu.async_copy(tab_hbm.at[srcv[slot]], rows[slot], sems[slot])

        @pl.when(nblk > 0)
        def _():
            stage_and_fire(0, 0)

        neg = jnp.full((16,), -1e30, f32)
        zero = jnp.zeros((16,), f32)
        one = jnp.ones((16,), f32)
        io16 = lax.iota(i32, 16)

        def allsum(v):
            for sh in (8, 4, 2, 1):
                v = v + v.at[io16 ^ sh].get(mode="promise_in_bounds",
                                            unique_indices=True)
            return v

        def finalize(cur_d, cnt, a0, a1, a2, mx_v, den_v, accs):
            dloc = cur_d - d0
            inv_v = 1.0 / jnp.full((16,), jnp.maximum(cnt, 1).astype(f32), f32)
            b0 = a0 * inv_v
            b1 = a1 * inv_v
            b2 = a2 * inv_v
            part = zero
            xld = [xlxr[dloc, pl.ds(kk * 16, 16)] for kk in range(4)]
            for kk in range(4):
                m = (xld[kk] + xlxr[dloc, pl.ds(64 + kk * 16, 16)]
                     + b0 * W0[kk] + b1 * W1[kk] + b2 * W2[kk])
                m = jnp.where(m > 0, m, 0.2 * m)
                part = part + m * AT[kk]
            lg_v = allsum(part)
            nmx = jnp.maximum(mx_v, lg_v)
            s = jnp.exp(mx_v - nmx)
            w = jnp.exp(lg_v - nmx)
            den = den_v * s + w
            for kk in range(4):
                out_k = (accs[kk] * s + w * xld[kk]) / den
                outb[pl.ds(dloc * HID + kk * 16, 16)] = out_k

        def edge_range(lo, hi, base, slot, carry):
            def logit_of(e):
                j = jnp.clip(e - base, 0, EB - 1)
                dnew = dstv[slot][pl.ds(j, 16)][0]
                ev = eav[slot][pl.ds(4 * j, 16)]
                c0 = ev[0]
                c1 = ev[1]
                c2 = ev[2]
                dloc = jnp.clip(dnew - d0, 0, NT - 1)
                part = zero
                rws = []
                for kk in range(4):
                    row = rows[slot][j, pl.ds(kk * 16, 16)]
                    rws.append(row)
                    m = (row + xlxr[dloc, pl.ds(64 + kk * 16, 16)]
                         + c0 * W0[kk] + c1 * W1[kk] + c2 * W2[kk])
                    m = jnp.where(m > 0, m, 0.2 * m)
                    part = part + m * AT[kk]
                return (allsum(part), dnew, c0, c1, c2) + tuple(rws)

            def update(c, pre, valid):
                (cur_d, cnt, a0, a1, a2, mx_v, den_v,
                 ac0, ac1, ac2, ac3) = c
                accs = (ac0, ac1, ac2, ac3)
                (lg_v, dnew, c0, c1, c2, r0, r1, r2, r3) = pre
                rws = (r0, r1, r2, r3)
                bnd = dnew != cur_d

                @pl.when(bnd & (cur_d >= 0) & valid)
                def _():
                    finalize(cur_d, cnt, a0, a1, a2, mx_v, den_v, accs)

                cnt = jnp.where(bnd, 1, cnt + 1)
                a0 = jnp.where(bnd, c0, a0 + c0)
                a1 = jnp.where(bnd, c1, a1 + c1)
                a2 = jnp.where(bnd, c2, a2 + c2)
                mx_v = jnp.where(bnd, neg, mx_v)
                den_v = jnp.where(bnd, zero, den_v)
                accs = tuple(jnp.where(bnd, zero, a) for a in accs)
                nmx = jnp.maximum(mx_v, lg_v)
                t = jnp.exp(jnp.minimum(mx_v, lg_v) - nmx)
                gt = lg_v > mx_v
                s = jnp.where(gt, t, one)
                w = jnp.where(gt, one, t)
                den_v = den_v * s + w
                accs = tuple(accs[kk] * s + w * rws[kk] for kk in range(4))
                return (dnew, cnt, a0, a1, a2, nmx, den_v) + accs

            pre0 = logit_of(lo)

            def edge_body(e, cp):
                c, pre = cp
                c = update(c, pre, True)
                return c, logit_of(e + 1)

            c_fin, pre_last = plsc.parallel_loop(
                lo, jnp.maximum(hi - 1, lo), unroll=2,
                carry=(carry, pre0))(edge_body)
            c_done = update(c_fin, pre_last, hi > lo)
            nonempty = hi > lo
            return tuple(
                jnp.where(nonempty, cd, cf)
                for cd, cf in zip(c_done, c_fin))

        carry0 = (i32(-1), i32(0), f32(0), f32(0), f32(0),
                  neg, zero, zero, zero, zero, zero)

        def pair_body(bp, carry):
            for phase in range(2):
                blk = 2 * bp + phase
                slot = phase
                base = e0al + blk * EB

                @pl.when(blk + 1 < nblk)
                def _():
                    stage_and_fire(blk + 1, 1 - slot)

                @pl.when(blk < nblk)
                def _():
                    pltpu.make_async_copy(tab_hbm.at[srcv[slot]],
                                          rows[slot], sems[slot]).wait()

                lo = jnp.maximum(e0, base)
                hi = jnp.minimum(e1, base + EB)
                hi = jnp.maximum(hi, lo)
                carry = edge_range(lo, hi, base, slot, carry)
            return carry

        npairs = (nblk + 1) // 2
        carry = lax.fori_loop(0, npairs, pair_body, carry0)

        (cur_d, cnt, a0, a1, a2, mx_v, den_v, ac0, ac1, ac2, ac3) = carry

        @pl.when(cur_d >= 0)
        def _():
            finalize(cur_d, cnt, a0, a1, a2, mx_v, den_v,
                     (ac0, ac1, ac2, ac3))

        pltpu.sync_copy(outb, out_hbm.at[pl.ds(pl.multiple_of(d0 * HID, 8), NT * HID)])

    return k(tab, src_s, dst_s, ea4, tp, par)


def kernel(x, edge_index, edge_attr, batch, params, fc_w, fc_b):
    E = edge_index.shape[1]
    src = edge_index[0].astype(i32)
    dst = edge_index[1].astype(i32)

    perm = jnp.argsort(dst)
    src_s = src[perm]
    dst_s = dst[perm]
    ea_s = edge_attr[perm]

    EPAD = E + 2 * EB
    src_p = jnp.zeros((EPAD,), i32).at[:E].set(src_s)
    dst_p = jnp.full((EPAD,), -7, i32).at[:E].set(dst_s)
    ea_p = jnp.zeros((EPAD, 4), f32).at[:E, :3].set(ea_s).reshape(-1)

    bounds = jnp.minimum(jnp.arange(33, dtype=i32) * NT, N)
    tp = jnp.searchsorted(dst_s, bounds).astype(i32)
    tp = jnp.concatenate([tp, jnp.full((15,), E, i32)])

    x_pad = jnp.zeros((NPAD, HID), f32).at[:N, :x.shape[1]].set(x)
    batch_pad = jnp.full((1, NPAD), NG, i32).at[0, :N].set(batch.astype(i32))

    proj0 = pl.pallas_call(
        _proj0_body,
        out_shape=jax.ShapeDtypeStruct((NPAD, 2 * HID), f32),
    )
    projr = pl.pallas_call(
        _proj_relu_body,
        out_shape=jax.ShapeDtypeStruct((NPAD, 2 * HID), f32),
    )
    final = pl.pallas_call(
        _final_body,
        out_shape=jax.ShapeDtypeStruct((NG, fc_w.shape[1]), f32),
    )

    prev = None
    for li, p in enumerate(params):
        din = p['Wl'].shape[0]
        wl = jnp.zeros((HID, HID), f32).at[:din].set(p['Wl'])
        wr = jnp.zeros((HID, HID), f32).at[:din].set(p['Wr'])
        w2 = jnp.concatenate([wl, wr], axis=1)
        b2 = jnp.concatenate([p['bl'], p['br']])
        if li == 0:
            tab = proj0(x_pad, w2, b2)
        else:
            tab = projr(prev, params[li - 1]['bias'], w2, b2)
        par = jnp.concatenate([p['We'], p['att'][None, :]], axis=0)
        out_flat = _sc_layer(tab, src_p, dst_p, ea_p, tp, par)
        prev = out_flat.reshape(NPAD, HID)

    return final(prev, params[-1]['bias'], batch_pad, fc_w, fc_b)


# speedup vs baseline: 1.1380x; 1.0155x over previous
import functools

import jax
import jax.numpy as jnp
from jax import lax
from jax.experimental import pallas as pl
from jax.experimental.pallas import tpu as pltpu
from jax.experimental.pallas import tpu_sc as plsc

f32 = jnp.float32
i32 = jnp.int32

N = 10000
NTILES = 32
NT = 320
NPAD = NTILES * NT
HID = 64
EB = 128
NG = 64

_HIGH = jax.lax.Precision.HIGHEST


def _proj0_body(x_ref, w_ref, b_ref, out_ref):
    out_ref[...] = jnp.dot(x_ref[...], w_ref[...], precision=_HIGH,
                           preferred_element_type=f32) + b_ref[...]


def _proj_relu_body(prev_ref, bias_ref, w_ref, b_ref, out_ref):
    h = jnp.maximum(prev_ref[...] + bias_ref[...], 0.0)
    out_ref[...] = jnp.dot(h, w_ref[...], precision=_HIGH,
                           preferred_element_type=f32) + b_ref[...]


def _final_body(prev_ref, bias_ref, batch_ref, fcw_ref, fcb_ref, out_ref):
    h = prev_ref[...] + bias_ref[...]
    g = lax.broadcasted_iota(i32, (NG, NPAD), 0)
    oht = (batch_ref[...] == g).astype(f32)
    pooled = jnp.dot(oht, h, precision=_HIGH, preferred_element_type=f32)
    out_ref[...] = jnp.dot(pooled, fcw_ref[...], precision=_HIGH,
                           preferred_element_type=f32) + fcb_ref[...]


def _sc_layer(tab, src_s, ea4, tp, par):
    mesh = plsc.VectorSubcoreMesh(core_axis_name="c", subcore_axis_name="s")

    @functools.partial(
        pl.kernel,
        mesh=mesh,
        out_type=jax.ShapeDtypeStruct((NPAD * HID,), f32),
        scratch_types=[
            pltpu.VMEM((NT, 128), f32),
            pltpu.VMEM((EB, 128), f32),
            pltpu.VMEM((EB, 128), f32),
            pltpu.VMEM((128,), i32),
            pltpu.VMEM((128,), i32),
            pltpu.VMEM((4 * EB + 16,), f32),
            pltpu.VMEM((4 * EB + 16,), f32),
            pltpu.VMEM((4, 64), f32),
            pltpu.VMEM((48,), i32),
            pltpu.VMEM((NT * HID,), f32),
            pltpu.SemaphoreType.DMA,
            pltpu.SemaphoreType.DMA,
        ],
    )
    def k(tab_hbm, src_hbm, ea_hbm, tp_hbm, par_hbm, out_hbm,
          xlxr, rows0, rows1, srcv0, srcv1, eav0, eav1,
          parv, tpv, outb, semA, semB):
        cid = lax.axis_index("c")
        sid = lax.axis_index("s")
        wid = sid * 2 + cid
        d0 = pl.multiple_of(wid * NT, 8)

        pltpu.sync_copy(tp_hbm, tpv)
        pltpu.sync_copy(par_hbm, parv)
        pltpu.sync_copy(tab_hbm.at[pl.ds(d0, NT), :], xlxr)

        tpw = tpv[pl.ds(wid, 16)]
        e0 = tpw[0]
        e1 = tpw[1]
        e0al = pl.multiple_of((e0 // 8) * 8, 8)
        nblk = (e1 - e0al + EB - 1) // EB

        W0 = [parv[0, pl.ds(kk * 16, 16)] for kk in range(4)]
        W1 = [parv[1, pl.ds(kk * 16, 16)] for kk in range(4)]
        W2 = [parv[2, pl.ds(kk * 16, 16)] for kk in range(4)]
        AT = [parv[3, pl.ds(kk * 16, 16)] for kk in range(4)]

        def init_body(d, c):
            for kk in range(4):
                outb[pl.ds(d * HID + kk * 16, 16)] = xlxr[d, pl.ds(kk * 16, 16)]
            return c
        lax.fori_loop(0, NT, init_body, i32(0))

        rows = (rows0, rows1)
        srcv = (srcv0, srcv1)
        eav = (eav0, eav1)
        sems = (semA, semB)

        def stage_and_fire(blk, slot):
            base = pl.multiple_of(e0al + blk * EB, 8)
            pltpu.sync_copy(src_hbm.at[pl.ds(base, EB)], srcv[slot])
            pltpu.sync_copy(ea_hbm.at[pl.ds(base * 4, 4 * EB)],
                            eav[slot].at[pl.ds(0, 4 * EB)])
            pltpu.async_copy(tab_hbm.at[srcv[slot]], rows[slot], sems[slot])

        @pl.when(nblk > 0)
        def _():
            stage_and_fire(0, 0)

        neg = jnp.full((16,), -1e30, f32)
        zero = jnp.zeros((16,), f32)
        one = jnp.ones((16,), f32)
        io16 = lax.iota(i32, 16)

        def allsum(v):
            for sh in (8, 4, 2, 1):
                v = v + v.at[io16 ^ sh].get(mode="promise_in_bounds",
                                            unique_indices=True)
            return v

        def finalize(cur_d, cnt, a0, a1, a2, mx_v, den_v, accs):
            dloc = cur_d - d0
            inv_v = 1.0 / jnp.full((16,), jnp.maximum(cnt, 1).astype(f32), f32)
            b0 = a0 * inv_v
            b1 = a1 * inv_v
            b2 = a2 * inv_v
            part = zero
            xld = [xlxr[dloc, pl.ds(kk * 16, 16)] for kk in range(4)]
            for kk in range(4):
                m = (xld[kk] + xlxr[dloc, pl.ds(64 + kk * 16, 16)]
                     + b0 * W0[kk] + b1 * W1[kk] + b2 * W2[kk])
                m = jnp.where(m > 0, m, 0.2 * m)
                part = part + m * AT[kk]
            lg_v = allsum(part)
            nmx = jnp.maximum(mx_v, lg_v)
            s = jnp.exp(mx_v - nmx)
            w = jnp.exp(lg_v - nmx)
            den = den_v * s + w
            for kk in range(4):
                out_k = (accs[kk] * s + w * xld[kk]) / den
                outb[pl.ds(dloc * HID + kk * 16, 16)] = out_k

        def edge_range(lo, hi, base, slot, carry):
            def logit_of(e):
                j = jnp.clip(e - base, 0, EB - 1)
                ev = eav[slot][pl.ds(4 * j, 16)]
                c0 = ev[0]
                c1 = ev[1]
                c2 = ev[2]
                dnew = ev[3].astype(i32)
                dloc = jnp.clip(dnew - d0, 0, NT - 1)
                part = zero
                rws = []
                for kk in range(4):
                    row = rows[slot][j, pl.ds(kk * 16, 16)]
                    rws.append(row)
                    m = (row + xlxr[dloc, pl.ds(64 + kk * 16, 16)]
                         + c0 * W0[kk] + c1 * W1[kk] + c2 * W2[kk])
                    m = jnp.where(m > 0, m, 0.2 * m)
                    part = part + m * AT[kk]
                return (allsum(part), dnew, c0, c1, c2) + tuple(rws)

            def update(c, pre, valid):
                (cur_d, cnt, a0, a1, a2, mx_v, den_v,
                 ac0, ac1, ac2, ac3) = c
                accs = (ac0, ac1, ac2, ac3)
                (lg_v, dnew, c0, c1, c2, r0, r1, r2, r3) = pre
                rws = (r0, r1, r2, r3)
                bnd = dnew != cur_d

                @pl.when(bnd & (cur_d >= 0) & valid)
                def _():
                    finalize(cur_d, cnt, a0, a1, a2, mx_v, den_v, accs)

                cnt = jnp.where(bnd, 1, cnt + 1)
                a0 = jnp.where(bnd, c0, a0 + c0)
                a1 = jnp.where(bnd, c1, a1 + c1)
                a2 = jnp.where(bnd, c2, a2 + c2)
                mx_v = jnp.where(bnd, neg, mx_v)
                den_v = jnp.where(bnd, zero, den_v)
                accs = tuple(jnp.where(bnd, zero, a) for a in accs)
                nmx = jnp.maximum(mx_v, lg_v)
                t = jnp.exp(jnp.minimum(mx_v, lg_v) - nmx)
                gt = lg_v > mx_v
                s = jnp.where(gt, t, one)
                w = jnp.where(gt, one, t)
                den_v = den_v * s + w
                accs = tuple(accs[kk] * s + w * rws[kk] for kk in range(4))
                return (dnew, cnt, a0, a1, a2, nmx, den_v) + accs

            pre0 = logit_of(lo)

            def edge_body(e, cp):
                c, pre = cp
                c = update(c, pre, True)
                return c, logit_of(e + 1)

            c_fin, pre_last = plsc.parallel_loop(
                lo, jnp.maximum(hi - 1, lo), unroll=4,
                carry=(carry, pre0))(edge_body)
            c_done = update(c_fin, pre_last, hi > lo)
            nonempty = hi > lo
            return tuple(
                jnp.where(nonempty, cd, cf)
                for cd, cf in zip(c_done, c_fin))

        carry0 = (i32(-1), i32(0), f32(0), f32(0), f32(0),
                  neg, zero, zero, zero, zero, zero)

        def pair_body(bp, carry):
            for phase in range(2):
                blk = 2 * bp + phase
                slot = phase
                base = e0al + blk * EB

                @pl.when(blk + 1 < nblk)
                def _():
                    stage_and_fire(blk + 1, 1 - slot)

                @pl.when(blk < nblk)
                def _():
                    pltpu.make_async_copy(tab_hbm.at[srcv[slot]],
                                          rows[slot], sems[slot]).wait()

                lo = jnp.maximum(e0, base)
                hi = jnp.minimum(e1, base + EB)
                hi = jnp.maximum(hi, lo)
                carry = edge_range(lo, hi, base, slot, carry)
            return carry

        npairs = (nblk + 1) // 2
        carry = lax.fori_loop(0, npairs, pair_body, carry0)

        (cur_d, cnt, a0, a1, a2, mx_v, den_v, ac0, ac1, ac2, ac3) = carry

        @pl.when(cur_d >= 0)
        def _():
            finalize(cur_d, cnt, a0, a1, a2, mx_v, den_v,
                     (ac0, ac1, ac2, ac3))

        pltpu.sync_copy(outb, out_hbm.at[pl.ds(pl.multiple_of(d0 * HID, 8), NT * HID)])

    return k(tab, src_s, ea4, tp, par)


def kernel(x, edge_index, edge_attr, batch, params, fc_w, fc_b):
    E = edge_index.shape[1]
    src = edge_index[0].astype(i32)
    dst = edge_index[1].astype(i32)

    perm = jnp.argsort(dst)
    src_s = src[perm]
    dst_s = dst[perm]
    ea_s = edge_attr[perm]

    EPAD = E + 2 * EB
    src_p = jnp.zeros((EPAD,), i32).at[:E].set(src_s)
    ea_p = (jnp.zeros((EPAD, 4), f32).at[:E, :3].set(ea_s)
            .at[:E, 3].set(dst_s.astype(f32)).reshape(-1))

    bounds = jnp.minimum(jnp.arange(33, dtype=i32) * NT, N)
    tp = jnp.searchsorted(dst_s, bounds).astype(i32)
    tp = jnp.concatenate([tp, jnp.full((15,), E, i32)])

    x_pad = jnp.zeros((NPAD, HID), f32).at[:N, :x.shape[1]].set(x)
    batch_pad = jnp.full((1, NPAD), NG, i32).at[0, :N].set(batch.astype(i32))

    proj0 = pl.pallas_call(
        _proj0_body,
        out_shape=jax.ShapeDtypeStruct((NPAD, 2 * HID), f32),
    )
    projr = pl.pallas_call(
        _proj_relu_body,
        out_shape=jax.ShapeDtypeStruct((NPAD, 2 * HID), f32),
    )
    final = pl.pallas_call(
        _final_body,
        out_shape=jax.ShapeDtypeStruct((NG, fc_w.shape[1]), f32),
    )

    prev = None
    for li, p in enumerate(params):
        din = p['Wl'].shape[0]
        wl = jnp.zeros((HID, HID), f32).at[:din].set(p['Wl'])
        wr = jnp.zeros((HID, HID), f32).at[:din].set(p['Wr'])
        w2 = jnp.concatenate([wl, wr], axis=1)
        b2 = jnp.concatenate([p['bl'], p['br']])
        if li == 0:
            tab = proj0(x_pad, w2, b2)
        else:
            tab = projr(prev, params[li - 1]['bias'], w2, b2)
        par = jnp.concatenate([p['We'], p['att'][None, :]], axis=0)
        out_flat = _sc_layer(tab, src_p, ea_p, tp, par)
        prev = out_flat.reshape(NPAD, HID)

    return final(prev, params[-1]['bias'], batch_pad, fc_w, fc_b)
